# Initial kernel scaffold; baseline (speedup 1.0000x reference)
import jax
import jax.numpy as jnp
from jax.experimental import pallas as pl


def kernel(x, edge_index, W1, att_src1, att_dst1, b1, W2, att_src2, att_dst2, b2):
    raise NotImplementedError("write your pallas kernel here")


# speedup vs baseline: 1.0903x; 1.0903x over previous
import functools

import jax
import jax.numpy as jnp
from jax.experimental import pallas as pl

N = 10000
E = 160000
F_IN = 256
HID = 256
HEADS = 4
CLASSES = 64


def _mm_body(x_ref, w_ref, o_ref):
    o_ref[...] = jnp.dot(x_ref[...], w_ref[...],
                         preferred_element_type=jnp.float32)


def _matmul(x, w, blk_m=1000):
    m, k = x.shape
    k2, n = w.shape
    grid = (m // blk_m,)
    return pl.pallas_call(
        _mm_body,
        grid=grid,
        in_specs=[
            pl.BlockSpec((blk_m, k), lambda i: (i, 0)),
            pl.BlockSpec((k, n), lambda i: (0, 0)),
        ],
        out_specs=pl.BlockSpec((blk_m, n), lambda i: (i, 0)),
        out_shape=jax.ShapeDtypeStruct((m, n), jnp.float32),
    )(x, w)


def _gat_conv(x, ei, W, att_src, att_dst, bias, heads, out_ch):
    n = x.shape[0]
    src, dst = ei[0], ei[1]
    h = _matmul(x, W).reshape(n, heads, out_ch)
    a_src = (h * att_src).sum(-1)
    a_dst = (h * att_dst).sum(-1)
    alpha = a_src[src] + a_dst[dst]
    alpha = jnp.where(alpha >= 0, alpha, 0.2 * alpha)
    ex = jnp.exp(alpha)
    denom = jax.ops.segment_sum(ex, dst, num_segments=n)
    w = ex / (denom[dst] + 1e-16)
    out = jax.ops.segment_sum(h[src] * w[:, :, None], dst, num_segments=n)
    return out.reshape(n, heads * out_ch) + bias


def kernel(x, edge_index, W1, att_src1, att_dst1, b1, W2, att_src2,
           att_dst2, b2):
    loop = jnp.arange(N, dtype=edge_index.dtype)
    ei = jnp.concatenate([edge_index, jnp.stack([loop, loop])], axis=1)
    h = _gat_conv(x, ei, W1, att_src1, att_dst1, b1, HEADS, HID)
    h = jax.nn.relu(h)
    out = _gat_conv(h, ei, W2, att_src2, att_dst2, b2, 1, CLASSES)
    return jax.nn.log_softmax(out, axis=1)


# speedup vs baseline: 4.4562x; 4.0870x over previous
import jax
import jax.numpy as jnp
from jax import lax
from jax.experimental import pallas as pl
from jax.experimental.pallas import tpu as pltpu
from jax.experimental.pallas import tpu_sc as plsc

N = 10000
E_RAW = 160000
E_TOT = E_RAW + N
NW = 32
NC = 2
NS = 16
L = 16
EPW = 5376
E_P = NW * EPW
HEADS = 4
HID = 256
D1 = HEADS * HID
CLASSES = 64
CPAD = 128

DBLK1 = 64
NBLK1 = -(-N // DBLK1)
R1 = -(-NBLK1 // NW)
NROW1 = NBLK1 * DBLK1

DBLK2 = 320
NROW2 = NW * DBLK2

CH_B = 1024
NCH_B = E_P // CH_B
BUF = 1536
FLUSH_AT = BUF - L
DST_PAD = 16383


def _mesh():
    return plsc.VectorSubcoreMesh(core_axis_name="c", subcore_axis_name="s",
                                  num_cores=NC, num_subcores=NS)


def _wid():
    return lax.axis_index("s") * NC + lax.axis_index("c")


def _iota():
    return lax.iota(jnp.int32, L)


def _splat(vec, j):
    idx = jnp.full((L,), 0, jnp.int32) + j
    return vec[idx]


def _tc1_body(x_ref, w_ref, asd_ref, h_ref, a_ref):
    h = jnp.dot(x_ref[...], w_ref[...], preferred_element_type=jnp.float32)
    h_ref[...] = h
    a_ref[...] = jnp.dot(h, asd_ref[...], preferred_element_type=jnp.float32)


def _tc1(x, W1, Asd):
    return pl.pallas_call(
        _tc1_body,
        grid=(10,),
        in_specs=[
            pl.BlockSpec((1000, 256), lambda i: (i, 0)),
            pl.BlockSpec((256, D1), lambda i: (0, 0)),
            pl.BlockSpec((D1, 8), lambda i: (0, 0)),
        ],
        out_specs=[
            pl.BlockSpec((1000, D1), lambda i: (i, 0)),
            pl.BlockSpec((1000, 8), lambda i: (i, 0)),
        ],
        out_shape=[
            jax.ShapeDtypeStruct((N, D1), jnp.float32),
            jax.ShapeDtypeStruct((N, 8), jnp.float32),
        ],
    )(x, W1, Asd)


def _tc2_body(num_ref, b_ref, w_ref, att_ref, h_ref, a_ref):
    z = jnp.maximum(num_ref[...] + b_ref[...], 0.0)
    h = jnp.dot(z, w_ref[...], preferred_element_type=jnp.float32)
    h_ref[...] = h
    a_ref[...] = jnp.dot(h, att_ref[...], preferred_element_type=jnp.float32)


def _tc2(num1, b1, W2p, att2):
    return pl.pallas_call(
        _tc2_body,
        grid=(10,),
        in_specs=[
            pl.BlockSpec((1000, D1), lambda i: (i, 0)),
            pl.BlockSpec((1, D1), lambda i: (0, 0)),
            pl.BlockSpec((D1, CPAD), lambda i: (0, 0)),
            pl.BlockSpec((CPAD, 8), lambda i: (0, 0)),
        ],
        out_specs=[
            pl.BlockSpec((1000, CPAD), lambda i: (i, 0)),
            pl.BlockSpec((1000, 8), lambda i: (i, 0)),
        ],
        out_shape=[
            jax.ShapeDtypeStruct((N, CPAD), jnp.float32),
            jax.ShapeDtypeStruct((N, 8), jnp.float32),
        ],
    )(num1, b1, W2p, att2)


def _tc3_body(num_ref, b_ref, o_ref):
    z = num_ref[...] + b_ref[...]
    m = jnp.max(z, axis=1, keepdims=True)
    zs = z - m
    lse = jnp.log(jnp.sum(jnp.exp(zs), axis=1, keepdims=True))
    o_ref[...] = zs - lse


def _tc3(num2, b2):
    return pl.pallas_call(
        _tc3_body,
        grid=(10,),
        in_specs=[
            pl.BlockSpec((1000, CLASSES), lambda i: (i, 0)),
            pl.BlockSpec((1, CLASSES), lambda i: (0, 0)),
        ],
        out_specs=pl.BlockSpec((1000, CLASSES), lambda i: (i, 0)),
        out_shape=jax.ShapeDtypeStruct((N, CLASSES), jnp.float32),
    )(num2, b2)


def _pass_b(nheads, width, rwidth, dblk, nrounds):
    hd = width // nheads
    hvr = hd // L
    accw = dblk * width

    def body(src_hbm, dst_hbm, h_hbm, as_hbm, ad_hbm, num_hbm,
             acc, dacc, asv, adl, dstc, srcc, srcl, dlocl, exb,
             rowb, sem):
        w = _wid()
        pltpu.sync_copy(as_hbm, asv)

        def process_batch(cnt):
            zero_i = jnp.zeros((L,), jnp.int32)
            srcl[pl.ds(cnt, L)] = zero_i
            dlocl[pl.ds(cnt, L)] = zero_i
            nb = (cnt + L - 1) // L

            def b_body(bb, carry):
                idxs = srcl.at[pl.ds(bb * L, L)]
                pltpu.async_copy(h_hbm.at[idxs], rowb, sem).wait()
                sv = srcl[pl.ds(bb * L, L)]
                dlocv = dlocl[pl.ds(bb * L, L)]
                live = (bb * L + _iota()) < cnt
                for h in range(nheads):
                    av = plsc.load_gather(asv, [sv * nheads + h])
                    bv = plsc.load_gather(adl, [dlocv * nheads + h])
                    al = av + bv
                    al = jnp.where(al >= 0.0, al, 0.2 * al)
                    exb[h] = jnp.where(live, jnp.exp(al), 0.0)

                def j_body(j, c2):
                    dloc = _splat(dlocv, j)[0]
                    dvec = jnp.zeros((L,), jnp.float32)
                    for h in range(nheads):
                        s = _splat(exb[h], j)
                        dvec = jnp.where(_iota() == h, s, dvec)
                        for c in range(hvr):
                            o = dloc * width + h * hd + c * L
                            r = rowb[j, pl.ds(h * hd + c * L, L)]
                            plsc.addupdate(acc.at[pl.ds(o, L)], s * r)
                    plsc.addupdate(dacc.at[pl.ds(dloc * L, L)], dvec)
                    return c2

                lax.fori_loop(0, L, j_body, 0, unroll=False)
                return carry

            lax.fori_loop(0, nb, b_body, 0, unroll=False)

        def round_body(r, rcarry):
            blk = r * NW + w

            @pl.when(blk * dblk < N)
            def _round():
                d0 = blk * dblk
                pltpu.sync_copy(
                    ad_hbm.at[pl.ds(d0 * nheads, dblk * nheads)], adl)
                zf = jnp.zeros((L,), jnp.float32)

                def z_body(i, c):
                    acc[pl.ds(i * L, L)] = zf
                    return c
                lax.fori_loop(0, accw // L, z_body, 0, unroll=False)

                def zd_body(i, c):
                    dacc[pl.ds(i * L, L)] = zf
                    return c
                lax.fori_loop(0, dblk, zd_body, 0, unroll=False)

                def ch_body(t, cnt):
                    off = t * CH_B
                    pltpu.sync_copy(dst_hbm.at[pl.ds(off, CH_B)], dstc)
                    pltpu.sync_copy(src_hbm.at[pl.ds(off, CH_B)], srcc)

                    def g_body(g, cnt):
                        dv = dstc[pl.ds(g * L, L)]
                        m = (dv >= d0) & (dv < d0 + dblk)
                        nm = jnp.sum(m.astype(jnp.int32))

                        @pl.when(nm > 0)
                        def _store():
                            sv = srcc[pl.ds(g * L, L)]
                            plsc.store_compressed(
                                srcl.at[pl.ds(cnt, L)], sv, mask=m)
                            plsc.store_compressed(
                                dlocl.at[pl.ds(cnt, L)], dv - d0, mask=m)
                        cnt = cnt + nm

                        @pl.when(cnt >= FLUSH_AT)
                        def _flush():
                            process_batch(cnt)
                        return jnp.where(cnt >= FLUSH_AT, 0, cnt)

                    return lax.fori_loop(0, CH_B // L, g_body, cnt,
                                         unroll=False)

                cnt = lax.fori_loop(0, NCH_B, ch_body, 0, unroll=False)

                @pl.when(cnt > 0)
                def _tail():
                    process_batch(cnt)

                def w_body(d, c):
                    dvv = dacc[pl.ds(d * L, L)]
                    for h in range(nheads):
                        den = _splat(dvv, h)
                        rinv = 1.0 / (den + 1e-16)
                        for cc in range(hvr):
                            o = d * width + h * hd + cc * L
                            acc[pl.ds(o, L)] = acc[pl.ds(o, L)] * rinv
                    return c
                lax.fori_loop(0, dblk, w_body, 0, unroll=False)
                pltpu.sync_copy(acc,
                                num_hbm.at[pl.ds(blk * accw, accw)])
            return rcarry

        lax.fori_loop(0, nrounds, round_body, 0, unroll=False)

    scratch = [
        pltpu.VMEM((accw,), jnp.float32),
        pltpu.VMEM((dblk * L,), jnp.float32),
        pltpu.VMEM((N * nheads,), jnp.float32),
        pltpu.VMEM((dblk * nheads,), jnp.float32),
        pltpu.VMEM((CH_B,), jnp.int32),
        pltpu.VMEM((CH_B,), jnp.int32),
        pltpu.VMEM((BUF + L,), jnp.int32),
        pltpu.VMEM((BUF + L,), jnp.int32),
        pltpu.VMEM((nheads, L), jnp.float32),
        pltpu.VMEM((L, rwidth), jnp.float32),
        pltpu.SemaphoreType.DMA,
    ]
    nblk = -(-N // dblk) if nrounds > 1 else NW
    return pl.kernel(
        body,
        out_type=jax.ShapeDtypeStruct((nblk * accw,), jnp.float32),
        mesh=_mesh(),
        scratch_types=scratch,
        compiler_params=pltpu.CompilerParams(needs_layout_passes=False),
    )


def kernel(x, edge_index, W1, att_src1, att_dst1, b1, W2, att_src2,
           att_dst2, b2):
    loop = jnp.arange(N, dtype=edge_index.dtype)
    ei = jnp.concatenate([edge_index, jnp.stack([loop, loop])], axis=1)
    src = jnp.pad(ei[0], (0, E_P - E_TOT)).astype(jnp.int32)
    dst = jnp.pad(ei[1], (0, E_P - E_TOT),
                  constant_values=DST_PAD).astype(jnp.int32)

    eye4 = jnp.eye(HEADS, dtype=jnp.float32)
    As = (att_src1[0][:, :, None] * eye4[:, None, :]).reshape(D1, HEADS)
    Ad = (att_dst1[0][:, :, None] * eye4[:, None, :]).reshape(D1, HEADS)
    Asd = jnp.concatenate([As, Ad], axis=1)

    h1, a1 = _tc1(x, W1, Asd)
    a1s = a1[:, :HEADS].reshape(-1)
    a1d = a1[:, HEADS:].reshape(-1)

    num1 = _pass_b(HEADS, D1, D1, DBLK1, R1)(src, dst, h1, a1s, a1d)
    num1 = num1.reshape(NROW1, D1)[:N]

    W2p = jnp.pad(W2, ((0, 0), (0, CPAD - CLASSES)))
    att2 = jnp.concatenate(
        [jnp.pad(att_src2[0].reshape(CLASSES, 1),
                 ((0, CPAD - CLASSES), (0, 0))),
         jnp.pad(att_dst2[0].reshape(CLASSES, 1),
                 ((0, CPAD - CLASSES), (0, 0))),
         jnp.zeros((CPAD, 6), jnp.float32)], axis=1)
    h2, a2 = _tc2(num1, b1.reshape(1, D1), W2p, att2)
    a2s = a2[:, 0]
    a2d = a2[:, 1]

    num2 = _pass_b(1, CLASSES, CPAD, DBLK2, 1)(src, dst, h2, a2s, a2d)
    num2 = num2.reshape(NROW2, CLASSES)[:N]

    return _tc3(num2, b2.reshape(1, CLASSES))


# speedup vs baseline: 5.0052x; 1.1232x over previous
import jax
import jax.numpy as jnp
from jax import lax
from jax.experimental import pallas as pl
from jax.experimental.pallas import tpu as pltpu
from jax.experimental.pallas import tpu_sc as plsc

N = 10000
E_RAW = 160000
E_TOT = E_RAW + N
NW = 32
NC = 2
NS = 16
L = 16
EPW = 5376
E_P = NW * EPW
HEADS = 4
HID = 256
D1 = HEADS * HID
CLASSES = 64
CPAD = 128

DBLK1 = 64
NBLK1 = -(-N // DBLK1)
R1 = -(-NBLK1 // NW)
NROW1 = NBLK1 * DBLK1

DBLK2 = 320
NROW2 = NW * DBLK2

CH_B = 1024
NCH_B = E_P // CH_B
BUF = 1536
FLUSH_AT = BUF - L
DST_PAD = 16383


def _mesh():
    return plsc.VectorSubcoreMesh(core_axis_name="c", subcore_axis_name="s",
                                  num_cores=NC, num_subcores=NS)


def _wid():
    return lax.axis_index("s") * NC + lax.axis_index("c")


def _iota():
    return lax.iota(jnp.int32, L)


def _splat(vec, j):
    idx = jnp.full((L,), 0, jnp.int32) + j
    return vec[idx]


def _tc1_body(x_ref, w_ref, asd_ref, h_ref, a_ref):
    h = jnp.dot(x_ref[...], w_ref[...], preferred_element_type=jnp.float32)
    a8 = jnp.dot(h, asd_ref[...], preferred_element_type=jnp.float32)
    az = jnp.pad(a8[:, :HEADS], ((0, 0), (0, 128 - HEADS)))
    h_ref[...] = jnp.concatenate([h, az], axis=1)
    a_ref[...] = a8


def _tc1(x, W1, Asd):
    return pl.pallas_call(
        _tc1_body,
        grid=(10,),
        in_specs=[
            pl.BlockSpec((1000, 256), lambda i: (i, 0)),
            pl.BlockSpec((256, D1), lambda i: (0, 0)),
            pl.BlockSpec((D1, 8), lambda i: (0, 0)),
        ],
        out_specs=[
            pl.BlockSpec((1000, D1 + 128), lambda i: (i, 0)),
            pl.BlockSpec((1000, 8), lambda i: (i, 0)),
        ],
        out_shape=[
            jax.ShapeDtypeStruct((N, D1 + 128), jnp.float32),
            jax.ShapeDtypeStruct((N, 8), jnp.float32),
        ],
    )(x, W1, Asd)


def _tc2_body(num_ref, b_ref, w_ref, att_ref, h_ref, a_ref):
    z = jnp.maximum(num_ref[...] + b_ref[...], 0.0)
    h = jnp.dot(z, w_ref[...], preferred_element_type=jnp.float32)
    a8 = jnp.dot(h, att_ref[...], preferred_element_type=jnp.float32)
    emb = jnp.pad(a8[:, :1], ((0, 0), (CLASSES, CPAD - CLASSES - 1)))
    h_ref[...] = h + emb
    a_ref[...] = a8


def _tc2(num1, b1, W2p, att2):
    return pl.pallas_call(
        _tc2_body,
        grid=(10,),
        in_specs=[
            pl.BlockSpec((1000, D1), lambda i: (i, 0)),
            pl.BlockSpec((1, D1), lambda i: (0, 0)),
            pl.BlockSpec((D1, CPAD), lambda i: (0, 0)),
            pl.BlockSpec((CPAD, 8), lambda i: (0, 0)),
        ],
        out_specs=[
            pl.BlockSpec((1000, CPAD), lambda i: (i, 0)),
            pl.BlockSpec((1000, 8), lambda i: (i, 0)),
        ],
        out_shape=[
            jax.ShapeDtypeStruct((N, CPAD), jnp.float32),
            jax.ShapeDtypeStruct((N, 8), jnp.float32),
        ],
    )(num1, b1, W2p, att2)


def _tc3_body(num_ref, b_ref, o_ref):
    z = num_ref[...] + b_ref[...]
    m = jnp.max(z, axis=1, keepdims=True)
    zs = z - m
    lse = jnp.log(jnp.sum(jnp.exp(zs), axis=1, keepdims=True))
    o_ref[...] = zs - lse


def _tc3(num2, b2):
    return pl.pallas_call(
        _tc3_body,
        grid=(10,),
        in_specs=[
            pl.BlockSpec((1000, CLASSES), lambda i: (i, 0)),
            pl.BlockSpec((1, CLASSES), lambda i: (0, 0)),
        ],
        out_specs=pl.BlockSpec((1000, CLASSES), lambda i: (i, 0)),
        out_shape=jax.ShapeDtypeStruct((N, CLASSES), jnp.float32),
    )(num2, b2)


def _pass_b(nheads, width, rwidth, aoff, dblk, nrounds):
    hd = width // nheads
    hvr = hd // L
    accw = dblk * width

    def body(src_hbm, dst_hbm, h_hbm, ad_hbm, num_hbm,
             acc, dacc, adl, dstc, srcc, srcl, dlocl, rowb, sem):
        w = _wid()

        def process_batch(cnt):
            zero_i = jnp.zeros((L,), jnp.int32)
            srcl[pl.ds(cnt, L)] = zero_i
            dlocl[pl.ds(cnt, L)] = zero_i
            nb = (cnt + L - 1) // L

            def b_body(bb, carry):
                idxs = srcl.at[pl.ds(bb * L, L)]
                pltpu.async_copy(h_hbm.at[idxs], rowb, sem).wait()
                dlocv = dlocl[pl.ds(bb * L, L)]

                def j_body(j, c2):
                    dloc = _splat(dlocv, j)[0]
                    asl = rowb[j, pl.ds(aoff, L)]
                    adsl = adl[pl.ds(dloc * nheads, L)]
                    al = asl + adsl
                    al = jnp.where(al >= 0.0, al, 0.2 * al)
                    exv = jnp.exp(al)
                    lv = (_iota() < nheads) & (bb * L + j < cnt)
                    exv = jnp.where(lv, exv, 0.0)
                    plsc.addupdate(dacc.at[pl.ds(dloc * L, L)], exv)
                    for h in range(nheads):
                        s = _splat(exv, h)
                        for c in range(hvr):
                            o = dloc * width + h * hd + c * L
                            r = rowb[j, pl.ds(h * hd + c * L, L)]
                            plsc.addupdate(acc.at[pl.ds(o, L)], s * r)
                    return c2

                lax.fori_loop(0, L, j_body, 0, unroll=False)
                return carry

            lax.fori_loop(0, nb, b_body, 0, unroll=False)

        def round_body(r, rcarry):
            blk = r * NW + w

            @pl.when(blk * dblk < N)
            def _round():
                d0 = blk * dblk
                pltpu.sync_copy(
                    ad_hbm.at[pl.ds(d0 * nheads, dblk * nheads)],
                    adl.at[pl.ds(0, dblk * nheads)])
                zf = jnp.zeros((L,), jnp.float32)
                adl[pl.ds(dblk * nheads, L)] = zf

                def z_body(i, c):
                    acc[pl.ds(i * L, L)] = zf
                    return c
                lax.fori_loop(0, accw // L, z_body, 0, unroll=8)

                def zd_body(i, c):
                    dacc[pl.ds(i * L, L)] = zf
                    return c
                lax.fori_loop(0, dblk, zd_body, 0, unroll=False)

                def ch_body(t, cnt):
                    off = t * CH_B
                    pltpu.sync_copy(dst_hbm.at[pl.ds(off, CH_B)], dstc)
                    pltpu.sync_copy(src_hbm.at[pl.ds(off, CH_B)], srcc)

                    def g_body(g, cnt):
                        dv = dstc[pl.ds(g * L, L)]
                        m = (dv >= d0) & (dv < d0 + dblk)
                        nm = plsc.all_reduce_population_count(m)[0]

                        @pl.when(nm > 0)
                        def _store():
                            sv = srcc[pl.ds(g * L, L)]
                            plsc.store_compressed(
                                srcl.at[pl.ds(cnt, L)], sv, mask=m)
                            plsc.store_compressed(
                                dlocl.at[pl.ds(cnt, L)], dv - d0, mask=m)
                        cnt = cnt + nm

                        @pl.when(cnt >= FLUSH_AT)
                        def _flush():
                            process_batch(cnt)
                        return jnp.where(cnt >= FLUSH_AT, 0, cnt)

                    return lax.fori_loop(0, CH_B // L, g_body, cnt,
                                         unroll=4)

                cnt = lax.fori_loop(0, NCH_B, ch_body, 0, unroll=False)

                @pl.when(cnt > 0)
                def _tail():
                    process_batch(cnt)

                def w_body(d, c):
                    dvv = dacc[pl.ds(d * L, L)]
                    for h in range(nheads):
                        den = _splat(dvv, h)
                        rinv = 1.0 / (den + 1e-16)
                        for cc in range(hvr):
                            o = d * width + h * hd + cc * L
                            acc[pl.ds(o, L)] = acc[pl.ds(o, L)] * rinv
                    return c
                lax.fori_loop(0, dblk, w_body, 0, unroll=False)
                pltpu.sync_copy(acc,
                                num_hbm.at[pl.ds(blk * accw, accw)])
            return rcarry

        lax.fori_loop(0, nrounds, round_body, 0, unroll=False)

    scratch = [
        pltpu.VMEM((accw,), jnp.float32),
        pltpu.VMEM((dblk * L,), jnp.float32),
        pltpu.VMEM((dblk * nheads + L,), jnp.float32),
        pltpu.VMEM((CH_B,), jnp.int32),
        pltpu.VMEM((CH_B,), jnp.int32),
        pltpu.VMEM((BUF + L,), jnp.int32),
        pltpu.VMEM((BUF + L,), jnp.int32),
        pltpu.VMEM((L, rwidth), jnp.float32),
        pltpu.SemaphoreType.DMA,
    ]
    nblk = -(-N // dblk) if nrounds > 1 else NW
    return pl.kernel(
        body,
        out_type=jax.ShapeDtypeStruct((nblk * accw,), jnp.float32),
        mesh=_mesh(),
        scratch_types=scratch,
        compiler_params=pltpu.CompilerParams(needs_layout_passes=False),
    )


def kernel(x, edge_index, W1, att_src1, att_dst1, b1, W2, att_src2,
           att_dst2, b2):
    loop = jnp.arange(N, dtype=edge_index.dtype)
    ei = jnp.concatenate([edge_index, jnp.stack([loop, loop])], axis=1)
    src = jnp.pad(ei[0], (0, E_P - E_TOT)).astype(jnp.int32)
    dst = jnp.pad(ei[1], (0, E_P - E_TOT),
                  constant_values=DST_PAD).astype(jnp.int32)

    eye4 = jnp.eye(HEADS, dtype=jnp.float32)
    As = (att_src1[0][:, :, None] * eye4[:, None, :]).reshape(D1, HEADS)
    Ad = (att_dst1[0][:, :, None] * eye4[:, None, :]).reshape(D1, HEADS)
    Asd = jnp.concatenate([As, Ad], axis=1)

    h1x, a1 = _tc1(x, W1, Asd)
    a1d = a1[:, HEADS:].reshape(-1)

    num1 = _pass_b(HEADS, D1, D1 + 128, D1, DBLK1, R1)(src, dst, h1x, a1d)
    num1 = num1.reshape(NROW1, D1)[:N]

    W2p = jnp.pad(W2, ((0, 0), (0, CPAD - CLASSES)))
    att2 = jnp.concatenate(
        [jnp.pad(att_src2[0].reshape(CLASSES, 1),
                 ((0, CPAD - CLASSES), (0, 0))),
         jnp.pad(att_dst2[0].reshape(CLASSES, 1),
                 ((0, CPAD - CLASSES), (0, 0))),
         jnp.zeros((CPAD, 6), jnp.float32)], axis=1)
    h2x, a2 = _tc2(num1, b1.reshape(1, D1), W2p, att2)
    a2d = a2[:, 1]

    num2 = _pass_b(1, CLASSES, CPAD, CLASSES, DBLK2, 1)(src, dst, h2x, a2d)
    num2 = num2.reshape(NROW2, CLASSES)[:N]

    return _tc3(num2, b2.reshape(1, CLASSES))


# speedup vs baseline: 5.6216x; 1.1231x over previous
import jax
import jax.numpy as jnp
from jax import lax
from jax.experimental import pallas as pl
from jax.experimental.pallas import tpu as pltpu
from jax.experimental.pallas import tpu_sc as plsc

N = 10000
E_RAW = 160000
E_TOT = E_RAW + N
NW = 32
NC = 2
NS = 16
L = 16
EPW = 5376
E_P = NW * EPW
HEADS = 4
HID = 256
D1 = HEADS * HID
CLASSES = 64
CPAD = 128

DBLK1 = 96
NBLK1 = -(-N // DBLK1)
R1 = -(-NBLK1 // NW)
NROW1 = NBLK1 * DBLK1

DBLK2 = 320
NROW2 = NW * DBLK2

CH_B = 4096
NCH_B = E_P // CH_B
BUF = 1536
FLUSH_AT = BUF - L
DST_PAD = 16383


def _mesh():
    return plsc.VectorSubcoreMesh(core_axis_name="c", subcore_axis_name="s",
                                  num_cores=NC, num_subcores=NS)


def _wid():
    return lax.axis_index("s") * NC + lax.axis_index("c")


def _iota():
    return lax.iota(jnp.int32, L)


def _splat(vec, j):
    idx = jnp.full((L,), 0, jnp.int32) + j
    return vec[idx]


def _tc1_body(x_ref, w_ref, asd_ref, h_ref, a_ref):
    h = jnp.dot(x_ref[...], w_ref[...], preferred_element_type=jnp.float32)
    a8 = jnp.dot(h, asd_ref[...], preferred_element_type=jnp.float32)
    az = jnp.pad(a8[:, :HEADS], ((0, 0), (0, 128 - HEADS)))
    h_ref[...] = jnp.concatenate([h, az], axis=1)
    a_ref[...] = a8


def _tc1(x, W1, Asd):
    return pl.pallas_call(
        _tc1_body,
        grid=(10,),
        in_specs=[
            pl.BlockSpec((1000, 256), lambda i: (i, 0)),
            pl.BlockSpec((256, D1), lambda i: (0, 0)),
            pl.BlockSpec((D1, 8), lambda i: (0, 0)),
        ],
        out_specs=[
            pl.BlockSpec((1000, D1 + 128), lambda i: (i, 0)),
            pl.BlockSpec((1000, 8), lambda i: (i, 0)),
        ],
        out_shape=[
            jax.ShapeDtypeStruct((N, D1 + 128), jnp.float32),
            jax.ShapeDtypeStruct((N, 8), jnp.float32),
        ],
    )(x, W1, Asd)


def _tc2_body(num_ref, b_ref, w_ref, att_ref, h_ref, a_ref):
    z = jnp.maximum(num_ref[...] + b_ref[...], 0.0)
    h = jnp.dot(z, w_ref[...], preferred_element_type=jnp.float32)
    a8 = jnp.dot(h, att_ref[...], preferred_element_type=jnp.float32)
    emb = jnp.pad(a8[:, :1], ((0, 0), (CLASSES, CPAD - CLASSES - 1)))
    h_ref[...] = h + emb
    a_ref[...] = a8


def _tc2(num1, b1, W2p, att2):
    return pl.pallas_call(
        _tc2_body,
        grid=(10,),
        in_specs=[
            pl.BlockSpec((1000, D1), lambda i: (i, 0)),
            pl.BlockSpec((1, D1), lambda i: (0, 0)),
            pl.BlockSpec((D1, CPAD), lambda i: (0, 0)),
            pl.BlockSpec((CPAD, 8), lambda i: (0, 0)),
        ],
        out_specs=[
            pl.BlockSpec((1000, CPAD), lambda i: (i, 0)),
            pl.BlockSpec((1000, 8), lambda i: (i, 0)),
        ],
        out_shape=[
            jax.ShapeDtypeStruct((N, CPAD), jnp.float32),
            jax.ShapeDtypeStruct((N, 8), jnp.float32),
        ],
    )(num1, b1, W2p, att2)


def _tc3_body(num_ref, b_ref, o_ref):
    z = num_ref[...] + b_ref[...]
    m = jnp.max(z, axis=1, keepdims=True)
    zs = z - m
    lse = jnp.log(jnp.sum(jnp.exp(zs), axis=1, keepdims=True))
    o_ref[...] = zs - lse


def _tc3(num2, b2):
    return pl.pallas_call(
        _tc3_body,
        grid=(10,),
        in_specs=[
            pl.BlockSpec((1000, CLASSES), lambda i: (i, 0)),
            pl.BlockSpec((1, CLASSES), lambda i: (0, 0)),
        ],
        out_specs=pl.BlockSpec((1000, CLASSES), lambda i: (i, 0)),
        out_shape=jax.ShapeDtypeStruct((N, CLASSES), jnp.float32),
    )(num2, b2)


def _pass_b(nheads, width, rwidth, aoff, dblk, nrounds):
    hd = width // nheads
    hvr = hd // L
    accw = dblk * width

    def body(src_hbm, dst_hbm, h_hbm, ad_hbm, num_hbm,
             acc, dacc, adl, dstc, srcc, srcl, dlocl, rowb, sem):
        w = _wid()

        def process_batch(cnt):
            zero_i = jnp.zeros((L,), jnp.int32)
            srcl[pl.ds(cnt, L)] = zero_i
            dlocl[pl.ds(cnt, L)] = zero_i
            nb = (cnt + L - 1) // L

            def b_body(bb, carry):
                idxs = srcl.at[pl.ds(bb * L, L)]
                pltpu.async_copy(h_hbm.at[idxs], rowb, sem).wait()
                dlocv = dlocl[pl.ds(bb * L, L)]

                def j_body(j, c2):
                    dloc = _splat(dlocv, j)[0]
                    asl = rowb[j, pl.ds(aoff, L)]
                    adsl = adl[pl.ds(dloc * nheads, L)]
                    al = asl + adsl
                    al = jnp.where(al >= 0.0, al, 0.2 * al)
                    exv = jnp.exp(al)
                    lv = (_iota() < nheads) & (bb * L + j < cnt)
                    exv = jnp.where(lv, exv, 0.0)
                    plsc.addupdate(dacc.at[pl.ds(dloc * L, L)], exv)
                    for h in range(nheads):
                        s = _splat(exv, h)
                        for c in range(hvr):
                            o = dloc * width + h * hd + c * L
                            r = rowb[j, pl.ds(h * hd + c * L, L)]
                            plsc.addupdate(acc.at[pl.ds(o, L)], s * r)
                    return c2

                lax.fori_loop(0, L, j_body, 0, unroll=False)
                return carry

            lax.fori_loop(0, nb, b_body, 0, unroll=False)

        def round_body(r, rcarry):
            blk = r * NW + w

            @pl.when(blk * dblk < N)
            def _round():
                d0 = blk * dblk
                pltpu.sync_copy(
                    ad_hbm.at[pl.ds(d0 * nheads, dblk * nheads)],
                    adl.at[pl.ds(0, dblk * nheads)])
                zf = jnp.zeros((L,), jnp.float32)
                adl[pl.ds(dblk * nheads, L)] = zf

                def z_body(i, c):
                    acc[pl.ds(i * L, L)] = zf
                    return c
                lax.fori_loop(0, accw // L, z_body, 0, unroll=8)

                def zd_body(i, c):
                    dacc[pl.ds(i * L, L)] = zf
                    return c
                lax.fori_loop(0, dblk, zd_body, 0, unroll=False)

                def ch_body(t, cnt):
                    off = t * CH_B
                    pltpu.sync_copy(dst_hbm.at[pl.ds(off, CH_B)], dstc)
                    pltpu.sync_copy(src_hbm.at[pl.ds(off, CH_B)], srcc)

                    def g_body(g, cnt):
                        dv = dstc[pl.ds(g * L, L)]
                        m = (dv >= d0) & (dv < d0 + dblk)
                        nm = plsc.all_reduce_population_count(m)[0]
                        sv = srcc[pl.ds(g * L, L)]
                        plsc.store_compressed(
                            srcl.at[pl.ds(cnt, L)], sv, mask=m)
                        plsc.store_compressed(
                            dlocl.at[pl.ds(cnt, L)], dv - d0, mask=m)
                        cnt = cnt + nm

                        @pl.when(cnt >= FLUSH_AT)
                        def _flush():
                            process_batch(cnt)
                        return jnp.where(cnt >= FLUSH_AT, 0, cnt)

                    return lax.fori_loop(0, CH_B // L, g_body, cnt,
                                         unroll=4)

                cnt = lax.fori_loop(0, NCH_B, ch_body, 0, unroll=False)

                @pl.when(cnt > 0)
                def _tail():
                    process_batch(cnt)

                def w_body(d, c):
                    dvv = dacc[pl.ds(d * L, L)]
                    for h in range(nheads):
                        den = _splat(dvv, h)
                        rinv = 1.0 / (den + 1e-16)
                        for cc in range(hvr):
                            o = d * width + h * hd + cc * L
                            acc[pl.ds(o, L)] = acc[pl.ds(o, L)] * rinv
                    return c
                lax.fori_loop(0, dblk, w_body, 0, unroll=False)
                pltpu.sync_copy(acc,
                                num_hbm.at[pl.ds(blk * accw, accw)])
            return rcarry

        lax.fori_loop(0, nrounds, round_body, 0, unroll=False)

    scratch = [
        pltpu.VMEM((accw,), jnp.float32),
        pltpu.VMEM((dblk * L,), jnp.float32),
        pltpu.VMEM((dblk * nheads + L,), jnp.float32),
        pltpu.VMEM((CH_B,), jnp.int32),
        pltpu.VMEM((CH_B,), jnp.int32),
        pltpu.VMEM((BUF + L,), jnp.int32),
        pltpu.VMEM((BUF + L,), jnp.int32),
        pltpu.VMEM((L, rwidth), jnp.float32),
        pltpu.SemaphoreType.DMA,
    ]
    nblk = -(-N // dblk) if nrounds > 1 else NW
    return pl.kernel(
        body,
        out_type=jax.ShapeDtypeStruct((nblk * accw,), jnp.float32),
        mesh=_mesh(),
        scratch_types=scratch,
        compiler_params=pltpu.CompilerParams(needs_layout_passes=False),
    )


def kernel(x, edge_index, W1, att_src1, att_dst1, b1, W2, att_src2,
           att_dst2, b2):
    loop = jnp.arange(N, dtype=edge_index.dtype)
    ei = jnp.concatenate([edge_index, jnp.stack([loop, loop])], axis=1)
    src = jnp.pad(ei[0], (0, E_P - E_TOT)).astype(jnp.int32)
    dst = jnp.pad(ei[1], (0, E_P - E_TOT),
                  constant_values=DST_PAD).astype(jnp.int32)

    eye4 = jnp.eye(HEADS, dtype=jnp.float32)
    As = (att_src1[0][:, :, None] * eye4[:, None, :]).reshape(D1, HEADS)
    Ad = (att_dst1[0][:, :, None] * eye4[:, None, :]).reshape(D1, HEADS)
    Asd = jnp.concatenate([As, Ad], axis=1)

    h1x, a1 = _tc1(x, W1, Asd)
    a1d = a1[:, HEADS:].reshape(-1)

    num1 = _pass_b(HEADS, D1, D1 + 128, D1, DBLK1, R1)(src, dst, h1x, a1d)
    num1 = num1.reshape(NROW1, D1)[:N]

    W2p = jnp.pad(W2, ((0, 0), (0, CPAD - CLASSES)))
    att2 = jnp.concatenate(
        [jnp.pad(att_src2[0].reshape(CLASSES, 1),
                 ((0, CPAD - CLASSES), (0, 0))),
         jnp.pad(att_dst2[0].reshape(CLASSES, 1),
                 ((0, CPAD - CLASSES), (0, 0))),
         jnp.zeros((CPAD, 6), jnp.float32)], axis=1)
    h2x, a2 = _tc2(num1, b1.reshape(1, D1), W2p, att2)
    a2d = a2[:, 1]

    num2 = _pass_b(1, CLASSES, CPAD, CLASSES, DBLK2, 1)(src, dst, h2x, a2d)
    num2 = num2.reshape(NROW2, CLASSES)[:N]

    return _tc3(num2, b2.reshape(1, CLASSES))


# speedup vs baseline: 7.2786x; 1.2948x over previous
import jax
import jax.numpy as jnp
from jax import lax
from jax.experimental import pallas as pl
from jax.experimental.pallas import tpu as pltpu
from jax.experimental.pallas import tpu_sc as plsc

N = 10000
E_RAW = 160000
E_TOT = E_RAW + N
NW = 32
NC = 2
NS = 16
L = 16
EPW = 5376
E_P = NW * EPW
HEADS = 4
HID = 256
D1 = HEADS * HID
CLASSES = 64
CPAD = 128

DBLK1 = 80
NBLK1 = -(-N // DBLK1)
R1 = -(-NBLK1 // NW)
NROW1 = NBLK1 * DBLK1

DBLK2 = 320
NROW2 = NW * DBLK2

CH_B = 2048
NCH_B = E_P // CH_B
BUF = 1024
FLUSH_AT = BUF - L
DST_PAD = 16383


def _mesh():
    return plsc.VectorSubcoreMesh(core_axis_name="c", subcore_axis_name="s",
                                  num_cores=NC, num_subcores=NS)


def _wid():
    return lax.axis_index("s") * NC + lax.axis_index("c")


def _iota():
    return lax.iota(jnp.int32, L)


def _splat(vec, j):
    idx = jnp.full((L,), 0, jnp.int32) + j
    return vec[idx]


def _tc1_body(x_ref, w_ref, asd_ref, h_ref, a_ref):
    h = jnp.dot(x_ref[...], w_ref[...], preferred_element_type=jnp.float32)
    a8 = jnp.dot(h, asd_ref[...], preferred_element_type=jnp.float32)
    az = jnp.pad(a8[:, :HEADS], ((0, 0), (0, 128 - HEADS)))
    h_ref[...] = jnp.concatenate([h, az], axis=1)
    a_ref[...] = a8


def _tc1(x, W1, Asd):
    return pl.pallas_call(
        _tc1_body,
        grid=(10,),
        in_specs=[
            pl.BlockSpec((1000, 256), lambda i: (i, 0)),
            pl.BlockSpec((256, D1), lambda i: (0, 0)),
            pl.BlockSpec((D1, 8), lambda i: (0, 0)),
        ],
        out_specs=[
            pl.BlockSpec((1000, D1 + 128), lambda i: (i, 0)),
            pl.BlockSpec((1000, 8), lambda i: (i, 0)),
        ],
        out_shape=[
            jax.ShapeDtypeStruct((N, D1 + 128), jnp.float32),
            jax.ShapeDtypeStruct((N, 8), jnp.float32),
        ],
    )(x, W1, Asd)


def _tc2_body(num_ref, b_ref, w_ref, att_ref, h_ref, a_ref):
    z = jnp.maximum(num_ref[...] + b_ref[...], 0.0)
    h = jnp.dot(z, w_ref[...], preferred_element_type=jnp.float32)
    a8 = jnp.dot(h, att_ref[...], preferred_element_type=jnp.float32)
    emb = jnp.pad(a8[:, :1], ((0, 0), (CLASSES, CPAD - CLASSES - 1)))
    h_ref[...] = h + emb
    a_ref[...] = a8


def _tc2(num1, b1, W2p, att2):
    return pl.pallas_call(
        _tc2_body,
        grid=(10,),
        in_specs=[
            pl.BlockSpec((1000, D1), lambda i: (i, 0)),
            pl.BlockSpec((1, D1), lambda i: (0, 0)),
            pl.BlockSpec((D1, CPAD), lambda i: (0, 0)),
            pl.BlockSpec((CPAD, 8), lambda i: (0, 0)),
        ],
        out_specs=[
            pl.BlockSpec((1000, CPAD), lambda i: (i, 0)),
            pl.BlockSpec((1000, 8), lambda i: (i, 0)),
        ],
        out_shape=[
            jax.ShapeDtypeStruct((N, CPAD), jnp.float32),
            jax.ShapeDtypeStruct((N, 8), jnp.float32),
        ],
    )(num1, b1, W2p, att2)


def _tc3_body(num_ref, b_ref, o_ref):
    z = num_ref[...] + b_ref[...]
    m = jnp.max(z, axis=1, keepdims=True)
    zs = z - m
    lse = jnp.log(jnp.sum(jnp.exp(zs), axis=1, keepdims=True))
    o_ref[...] = zs - lse


def _tc3(num2, b2):
    return pl.pallas_call(
        _tc3_body,
        grid=(10,),
        in_specs=[
            pl.BlockSpec((1000, CLASSES), lambda i: (i, 0)),
            pl.BlockSpec((1, CLASSES), lambda i: (0, 0)),
        ],
        out_specs=pl.BlockSpec((1000, CLASSES), lambda i: (i, 0)),
        out_shape=jax.ShapeDtypeStruct((N, CLASSES), jnp.float32),
    )(num2, b2)


def _pass_b(nheads, width, rwidth, aoff, dblk, nrounds):
    hd = width // nheads
    hvr = hd // L
    accw = dblk * width

    def body(src_hbm, dst_hbm, h_hbm, ad_hbm, num_hbm,
             acc, dacc, adl, dstc, srcc, srcl, dlocl, rowb, rowb2,
             sem, sem2):
        w = _wid()

        def _start(bb, buf, sm):
            pltpu.async_copy(h_hbm.at[srcl.at[pl.ds(bb * L, L)]], buf, sm)

        def _wait(bb, buf, sm):
            pltpu.make_async_copy(h_hbm.at[srcl.at[pl.ds(bb * L, L)]],
                                  buf, sm).wait()

        def process_batch(cnt):
            zero_i = jnp.zeros((L,), jnp.int32)
            srcl[pl.ds(cnt, L)] = zero_i
            srcl[pl.ds(cnt + L, L)] = zero_i
            dlocl[pl.ds(cnt, L)] = zero_i
            dlocl[pl.ds(cnt + L, L)] = zero_i
            nb = (cnt + L - 1) // L

            def _process(bb, buf):
                dlocv = dlocl[pl.ds(bb * L, L)]

                def j_body(j, c2):
                    dloc = _splat(dlocv, j)[0]
                    asl = buf[j, pl.ds(aoff, L)]
                    adsl = adl[pl.ds(dloc * nheads, L)]
                    al = asl + adsl
                    al = jnp.where(al >= 0.0, al, 0.2 * al)
                    exv = jnp.exp(al)
                    lv = (_iota() < nheads) & (bb * L + j < cnt)
                    exv = jnp.where(lv, exv, 0.0)
                    plsc.addupdate(dacc.at[pl.ds(dloc * L, L)], exv)
                    for h in range(nheads):
                        s = _splat(exv, h)
                        for c in range(hvr):
                            o = dloc * width + h * hd + c * L
                            r = buf[j, pl.ds(h * hd + c * L, L)]
                            plsc.addupdate(acc.at[pl.ds(o, L)], s * r)
                    return c2

                lax.fori_loop(0, L, j_body, 0, unroll=False)

            nbp = 2 * ((nb + 1) // 2)
            _start(0, rowb, sem)

            def pair_body(k, carry):
                bb0 = 2 * k
                _start(bb0 + 1, rowb2, sem2)
                _wait(bb0, rowb, sem)
                _process(bb0, rowb)
                _start(bb0 + 2, rowb, sem)
                _wait(bb0 + 1, rowb2, sem2)
                _process(bb0 + 1, rowb2)
                return carry

            lax.fori_loop(0, nbp // 2 - 1, pair_body, 0, unroll=False)
            bb0 = nbp - 2
            _start(bb0 + 1, rowb2, sem2)
            _wait(bb0, rowb, sem)
            _process(bb0, rowb)
            _wait(bb0 + 1, rowb2, sem2)
            _process(bb0 + 1, rowb2)

        def round_body(r, rcarry):
            blk = r * NW + w

            @pl.when(blk * dblk < N)
            def _round():
                d0 = blk * dblk
                pltpu.sync_copy(
                    ad_hbm.at[pl.ds(d0 * nheads, dblk * nheads)],
                    adl.at[pl.ds(0, dblk * nheads)])
                zf = jnp.zeros((L,), jnp.float32)
                adl[pl.ds(dblk * nheads, L)] = zf

                def z_body(i, c):
                    acc[pl.ds(i * L, L)] = zf
                    return c
                lax.fori_loop(0, accw // L, z_body, 0, unroll=8)

                def zd_body(i, c):
                    dacc[pl.ds(i * L, L)] = zf
                    return c
                lax.fori_loop(0, dblk, zd_body, 0, unroll=False)

                def ch_body(t, cnt):
                    off = t * CH_B
                    pltpu.sync_copy(dst_hbm.at[pl.ds(off, CH_B)], dstc)
                    pltpu.sync_copy(src_hbm.at[pl.ds(off, CH_B)], srcc)

                    def g_body(g, cnt):
                        dv = dstc[pl.ds(g * L, L)]
                        m = (dv >= d0) & (dv < d0 + dblk)
                        nm = plsc.all_reduce_population_count(m)[0]
                        sv = srcc[pl.ds(g * L, L)]
                        plsc.store_compressed(
                            srcl.at[pl.ds(cnt, L)], sv, mask=m)
                        plsc.store_compressed(
                            dlocl.at[pl.ds(cnt, L)], dv - d0, mask=m)
                        cnt = cnt + nm

                        @pl.when(cnt >= FLUSH_AT)
                        def _flush():
                            process_batch(cnt)
                        return jnp.where(cnt >= FLUSH_AT, 0, cnt)

                    return lax.fori_loop(0, CH_B // L, g_body, cnt,
                                         unroll=4)

                cnt = lax.fori_loop(0, NCH_B, ch_body, 0, unroll=False)

                @pl.when(cnt > 0)
                def _tail():
                    process_batch(cnt)

                def w_body(d, c):
                    dvv = dacc[pl.ds(d * L, L)]
                    for h in range(nheads):
                        den = _splat(dvv, h)
                        rinv = 1.0 / (den + 1e-16)
                        for cc in range(hvr):
                            o = d * width + h * hd + cc * L
                            acc[pl.ds(o, L)] = acc[pl.ds(o, L)] * rinv
                    return c
                lax.fori_loop(0, dblk, w_body, 0, unroll=False)
                pltpu.sync_copy(acc,
                                num_hbm.at[pl.ds(blk * accw, accw)])
            return rcarry

        lax.fori_loop(0, nrounds, round_body, 0, unroll=False)

    scratch = [
        pltpu.VMEM((accw,), jnp.float32),
        pltpu.VMEM((dblk * L,), jnp.float32),
        pltpu.VMEM((dblk * nheads + L,), jnp.float32),
        pltpu.VMEM((CH_B,), jnp.int32),
        pltpu.VMEM((CH_B,), jnp.int32),
        pltpu.VMEM((BUF + 2 * L,), jnp.int32),
        pltpu.VMEM((BUF + 2 * L,), jnp.int32),
        pltpu.VMEM((L, rwidth), jnp.float32),
        pltpu.VMEM((L, rwidth), jnp.float32),
        pltpu.SemaphoreType.DMA,
        pltpu.SemaphoreType.DMA,
    ]
    nblk = -(-N // dblk) if nrounds > 1 else NW
    return pl.kernel(
        body,
        out_type=jax.ShapeDtypeStruct((nblk * accw,), jnp.float32),
        mesh=_mesh(),
        scratch_types=scratch,
        compiler_params=pltpu.CompilerParams(needs_layout_passes=False),
    )


def kernel(x, edge_index, W1, att_src1, att_dst1, b1, W2, att_src2,
           att_dst2, b2):
    loop = jnp.arange(N, dtype=edge_index.dtype)
    ei = jnp.concatenate([edge_index, jnp.stack([loop, loop])], axis=1)
    src = jnp.pad(ei[0], (0, E_P - E_TOT)).astype(jnp.int32)
    dst = jnp.pad(ei[1], (0, E_P - E_TOT),
                  constant_values=DST_PAD).astype(jnp.int32)

    eye4 = jnp.eye(HEADS, dtype=jnp.float32)
    As = (att_src1[0][:, :, None] * eye4[:, None, :]).reshape(D1, HEADS)
    Ad = (att_dst1[0][:, :, None] * eye4[:, None, :]).reshape(D1, HEADS)
    Asd = jnp.concatenate([As, Ad], axis=1)

    h1x, a1 = _tc1(x, W1, Asd)
    a1d = a1[:, HEADS:].reshape(-1)

    num1 = _pass_b(HEADS, D1, D1 + 128, D1, DBLK1, R1)(src, dst, h1x, a1d)
    num1 = num1.reshape(NROW1, D1)[:N]

    W2p = jnp.pad(W2, ((0, 0), (0, CPAD - CLASSES)))
    att2 = jnp.concatenate(
        [jnp.pad(att_src2[0].reshape(CLASSES, 1),
                 ((0, CPAD - CLASSES), (0, 0))),
         jnp.pad(att_dst2[0].reshape(CLASSES, 1),
                 ((0, CPAD - CLASSES), (0, 0))),
         jnp.zeros((CPAD, 6), jnp.float32)], axis=1)
    h2x, a2 = _tc2(num1, b1.reshape(1, D1), W2p, att2)
    a2d = a2[:, 1]

    num2 = _pass_b(1, CLASSES, CPAD, CLASSES, DBLK2, 1)(src, dst, h2x, a2d)
    num2 = num2.reshape(NROW2, CLASSES)[:N]

    return _tc3(num2, b2.reshape(1, CLASSES))
